# baseline (speedup 1.0000x reference)
import jax
import jax.numpy as jnp
from jax.experimental import pallas as pl
from jax.experimental.pallas import tpu as pltpu

_R = float(1.25**2)
_ROWS, _COLS = 4096, 512
_BROWS = 512
_HALF = _ROWS // 2
_GRID = _HALF // _BROWS


def _tc_body(pa_ref, pb_ref, ta_ref, tb_ref, out_ref, acc_ref):
    @pl.when(pl.program_id(0) == 0)
    def _():
        acc_ref[0] = 0.0
        acc_ref[1] = 0.0

    g = jnp.float32(0.0)
    n = jnp.float32(0.0)
    for p_ref, t_ref in ((pa_ref, ta_ref), (pb_ref, tb_ref)):
        p = p_ref[...]
        t = t_ref[...]
        good = (p < _R * t) & (t < _R * p)
        g += jnp.sum(good.astype(jnp.float32))
        n += jnp.sum((t > 0.0).astype(jnp.float32))
    acc_ref[0] += g
    acc_ref[1] += n

    @pl.when(pl.program_id(0) == _GRID - 1)
    def _():
        out_ref[0] = acc_ref[0] / acc_ref[1]


_tc_ratio = pl.pallas_call(
    _tc_body,
    grid=(_GRID,),
    in_specs=[
        pl.BlockSpec((_BROWS, _COLS), lambda i: (i, 0)),
        pl.BlockSpec((_BROWS, _COLS), lambda i: (i + _HALF // _BROWS, 0)),
        pl.BlockSpec((_BROWS, _COLS), lambda i: (i, 0)),
        pl.BlockSpec((_BROWS, _COLS), lambda i: (i + _HALF // _BROWS, 0)),
    ],
    out_specs=pl.BlockSpec(memory_space=pltpu.SMEM),
    out_shape=jax.ShapeDtypeStruct((1,), jnp.float32),
    scratch_shapes=[pltpu.SMEM((2,), jnp.float32)],
    compiler_params=pltpu.CompilerParams(
        dimension_semantics=("arbitrary",),
    ),
)


def kernel(pred, target):
    p = pred.reshape(_ROWS, _COLS)
    t = target.reshape(_ROWS, _COLS)
    return _tc_ratio(p, p, t, t)[0]


# speedup vs baseline: 1.0391x; 1.0391x over previous
import jax
import jax.numpy as jnp
from jax.experimental import pallas as pl
from jax.experimental.pallas import tpu as pltpu

_R = float(1.25**2)
_ROWS, _COLS = 4096, 512
_BROWS = 1024
_HALF = _ROWS // 2
_GRID = _HALF // _BROWS


def _tc_body(pa_ref, pb_ref, ta_ref, tb_ref, out_ref, acc_ref):
    @pl.when(pl.program_id(0) == 0)
    def _():
        acc_ref[0] = 0.0
        acc_ref[1] = 0.0

    g = jnp.float32(0.0)
    n = jnp.float32(0.0)
    for p_ref, t_ref in ((pa_ref, ta_ref), (pb_ref, tb_ref)):
        p = p_ref[...]
        t = t_ref[...]
        good = (p < _R * t) & (t < _R * p)
        g += jnp.sum(good.astype(jnp.float32))
        n += jnp.sum((t > 0.0).astype(jnp.float32))
    acc_ref[0] += g
    acc_ref[1] += n

    @pl.when(pl.program_id(0) == _GRID - 1)
    def _():
        out_ref[0] = acc_ref[0] / acc_ref[1]


_tc_ratio = pl.pallas_call(
    _tc_body,
    grid=(_GRID,),
    in_specs=[
        pl.BlockSpec((_BROWS, _COLS), lambda i: (i, 0)),
        pl.BlockSpec((_BROWS, _COLS), lambda i: (i + _HALF // _BROWS, 0)),
        pl.BlockSpec((_BROWS, _COLS), lambda i: (i, 0)),
        pl.BlockSpec((_BROWS, _COLS), lambda i: (i + _HALF // _BROWS, 0)),
    ],
    out_specs=pl.BlockSpec(memory_space=pltpu.SMEM),
    out_shape=jax.ShapeDtypeStruct((1,), jnp.float32),
    scratch_shapes=[pltpu.SMEM((2,), jnp.float32)],
    compiler_params=pltpu.CompilerParams(
        dimension_semantics=("arbitrary",),
    ),
)


def kernel(pred, target):
    p = pred.reshape(_ROWS, _COLS)
    t = target.reshape(_ROWS, _COLS)
    return _tc_ratio(p, p, t, t)[0]
